# baseline (speedup 1.0000x reference)
import jax
import jax.numpy as jnp
from jax.experimental import pallas as pl
from jax.experimental.pallas import tpu as pltpu

HIDDEN = 128
N_EDGES = 320000
N_X = 10000
GRID = 25
BE = N_EDGES // (2 * GRID)
BX = N_X // GRID


def _tree_sum8(a):
    rows = a.shape[0]
    if rows > 128 and rows % 128 == 0:
        a = a.reshape(rows // 128, 128, HIDDEN).sum(axis=0)
        rows = 128
    while rows > 8 and rows % 16 == 0:
        rows //= 2
        a = a[:rows] + a[rows:]
    if rows > 8:
        a = a.reshape(rows // 8, 8, HIDDEN).sum(axis=0)
    return a


def _gb_kernel(ea_ref, eb_ref, x_ref, g_ref, w1_ref, b1_ref, w2_ref, b2_ref,
               gamma_ref, beta_ref, out_ref, acc_ref):
    i = pl.program_id(0)

    @pl.when(i == 0)
    def _init():
        acc_ref[...] = jnp.zeros_like(acc_ref)

    acc_ref[0:8, :] += _tree_sum8(x_ref[...])
    acc_ref[8:16, :] += _tree_sum8(ea_ref[...])
    acc_ref[16:24, :] += _tree_sum8(eb_ref[...])

    @pl.when(i == GRID - 1)
    def _finish():
        sn = jnp.sum(acc_ref[0:8, :], axis=0, keepdims=True)
        se = jnp.sum(acc_ref[8:16, :] + acc_ref[16:24, :], axis=0, keepdims=True)
        g = g_ref[...]
        h = (jnp.dot(sn, w1_ref[0:HIDDEN, :], preferred_element_type=jnp.float32)
             + jnp.dot(se, w1_ref[HIDDEN:2 * HIDDEN, :], preferred_element_type=jnp.float32)
             + jnp.dot(g, w1_ref[2 * HIDDEN:3 * HIDDEN, :], preferred_element_type=jnp.float32)
             + b1_ref[...])
        h = jnp.maximum(h, 0.0)
        out = jnp.dot(h, w2_ref[...], preferred_element_type=jnp.float32) + b2_ref[...]
        mean = jnp.mean(out, axis=-1, keepdims=True)
        var = jnp.mean((out - mean) ** 2, axis=-1, keepdims=True)
        out_ref[...] = ((out - mean) * jax.lax.rsqrt(var + 1e-5)
                        * gamma_ref[...] + beta_ref[...])


def kernel(x, edge_attr_updated, global_attr, W1, b1, W2, b2, gamma, beta):
    b1r = b1.reshape(1, HIDDEN)
    b2r = b2.reshape(1, HIDDEN)
    gammar = gamma.reshape(1, HIDDEN)
    betar = beta.reshape(1, HIDDEN)

    const = lambda i: (0, 0)
    return pl.pallas_call(
        _gb_kernel,
        grid=(GRID,),
        in_specs=[
            pl.BlockSpec((BE, HIDDEN), lambda i: (2 * i, 0)),
            pl.BlockSpec((BE, HIDDEN), lambda i: (2 * i + 1, 0)),
            pl.BlockSpec((BX, HIDDEN), lambda i: (i, 0)),
            pl.BlockSpec((1, HIDDEN), const),
            pl.BlockSpec((3 * HIDDEN, HIDDEN), const),
            pl.BlockSpec((1, HIDDEN), const),
            pl.BlockSpec((HIDDEN, HIDDEN), const),
            pl.BlockSpec((1, HIDDEN), const),
            pl.BlockSpec((1, HIDDEN), const),
            pl.BlockSpec((1, HIDDEN), const),
        ],
        out_specs=pl.BlockSpec((1, HIDDEN), const),
        out_shape=jax.ShapeDtypeStruct((1, HIDDEN), jnp.float32),
        scratch_shapes=[pltpu.VMEM((24, HIDDEN), jnp.float32)],
        compiler_params=pltpu.CompilerParams(
            dimension_semantics=("arbitrary",),
        ),
    )(edge_attr_updated, edge_attr_updated, x, global_attr, W1, b1r, W2,
      b2r, gammar, betar)
